# Initial kernel scaffold; baseline (speedup 1.0000x reference)
import functools

import jax
import jax.numpy as jnp
from jax import lax
from jax.experimental import pallas as pl
from jax.experimental.pallas import tpu as pltpu
from jax.experimental.pallas import tpu_sc as plsc

DICT_SIZE = 1000000
EMBED_SIZE = 32
BATCH = 16384
FIELDS = 26

_INFO = plsc.get_sparse_core_info()
_NC = _INFO.num_cores
_NS = _INFO.num_subcores
_NW = _NC * _NS

_B = BATCH * FIELDS
_B_PER_W = _B // _NW
_CHUNK = 1664
_NCHUNKS = _B_PER_W // _CHUNK


def _make_gather():
  mesh = plsc.VectorSubcoreMesh(core_axis_name="c", subcore_axis_name="s")

  @functools.partial(
      pl.kernel,
      mesh=mesh,
      out_type=jax.ShapeDtypeStruct((_B, EMBED_SIZE), jnp.float32),
      scratch_types=[
          pltpu.VMEM((_B_PER_W,), jnp.int32),
          pltpu.VMEM((_CHUNK, EMBED_SIZE), jnp.float32),
          pltpu.VMEM((_CHUNK, EMBED_SIZE), jnp.float32),
          pltpu.SemaphoreType.DMA,
          pltpu.SemaphoreType.DMA,
      ],
  )
  def k(table_hbm, idx_hbm, out_hbm, idx_v, rows0, rows1, sem0, sem1):
    wid = lax.axis_index("s") * _NC + lax.axis_index("c")
    base = wid * _B_PER_W
    pltpu.sync_copy(idx_hbm.at[pl.ds(base, _B_PER_W)], idx_v)

    rows = (rows0, rows1)
    sems = (sem0, sem1)
    pltpu.async_copy(table_hbm.at[idx_v.at[pl.ds(0, _CHUNK)]], rows0, sem0)
    for c in range(_NCHUNKS):
      cur = c % 2
      nxt = (c + 1) % 2
      if c + 1 < _NCHUNKS:
        pltpu.async_copy(
            table_hbm.at[idx_v.at[pl.ds((c + 1) * _CHUNK, _CHUNK)]],
            rows[nxt], sems[nxt])
      pltpu.make_async_copy(
          table_hbm.at[idx_v.at[pl.ds(c * _CHUNK, _CHUNK)]],
          rows[cur], sems[cur]).wait()
      pltpu.sync_copy(rows[cur],
                      out_hbm.at[pl.ds(base + c * _CHUNK, _CHUNK)])

  return k


_gather = _make_gather()


def kernel(ids, table):
  flat = ids.reshape(_B).astype(jnp.int32)
  out = _gather(table, flat)
  return out.reshape(BATCH, FIELDS, EMBED_SIZE)


# speedup vs baseline: 1.5768x; 1.5768x over previous
import functools

import jax
import jax.numpy as jnp
from jax import lax
from jax.experimental import pallas as pl
from jax.experimental.pallas import tpu as pltpu
from jax.experimental.pallas import tpu_sc as plsc

DICT_SIZE = 1000000
EMBED_SIZE = 32
BATCH = 16384
FIELDS = 26

_INFO = plsc.get_sparse_core_info()
_NC = _INFO.num_cores
_NS = _INFO.num_subcores
_NW = _NC * _NS

_B = BATCH * FIELDS
_B_PER_W = _B // _NW
_CHUNK = 1664
_NCHUNKS = _B_PER_W // _CHUNK


def _make_gather():
  mesh = plsc.VectorSubcoreMesh(core_axis_name="c", subcore_axis_name="s")

  @functools.partial(
      pl.kernel,
      mesh=mesh,
      out_type=jax.ShapeDtypeStruct((_B, EMBED_SIZE), jnp.float32),
      compiler_params=pltpu.CompilerParams(use_tc_tiling_on_sc=False),
      scratch_types=[
          pltpu.VMEM((_B_PER_W,), jnp.int32),
          pltpu.VMEM((_CHUNK, EMBED_SIZE), jnp.float32),
          pltpu.VMEM((_CHUNK, EMBED_SIZE), jnp.float32),
          pltpu.SemaphoreType.DMA,
          pltpu.SemaphoreType.DMA,
      ],
  )
  def k(table_hbm, idx_hbm, out_hbm, idx_v, rows0, rows1, sem0, sem1):
    wid = lax.axis_index("s") * _NC + lax.axis_index("c")
    base = wid * _B_PER_W
    pltpu.sync_copy(idx_hbm.at[pl.ds(base, _B_PER_W)], idx_v)

    rows = (rows0, rows1)
    sems = (sem0, sem1)
    pltpu.async_copy(table_hbm.at[idx_v.at[pl.ds(0, _CHUNK)]], rows0, sem0)
    for c in range(_NCHUNKS):
      cur = c % 2
      nxt = (c + 1) % 2
      if c + 1 < _NCHUNKS:
        pltpu.async_copy(
            table_hbm.at[idx_v.at[pl.ds((c + 1) * _CHUNK, _CHUNK)]],
            rows[nxt], sems[nxt])
      pltpu.make_async_copy(
          table_hbm.at[idx_v.at[pl.ds(c * _CHUNK, _CHUNK)]],
          rows[cur], sems[cur]).wait()
      pltpu.sync_copy(rows[cur],
                      out_hbm.at[pl.ds(base + c * _CHUNK, _CHUNK)])

  return k


_gather = _make_gather()


def kernel(ids, table):
  flat = ids.reshape(_B).astype(jnp.int32)
  out = _gather(table, flat)
  return out.reshape(BATCH, FIELDS, EMBED_SIZE)
